# baseline (device time: 42689 ns/iter reference)
import jax
import jax.numpy as jnp
from jax import lax
from jax.experimental import pallas as pl
from jax.experimental.pallas import tpu as pltpu

N_DEV = 4


def _gelu(y):
    c = 0.7978845608028654
    return 0.5 * y * (1.0 + jnp.tanh(c * (y + 0.044715 * y * y * y)))


def kernel(x, w_mat):
    m_per, k = x.shape
    _, n_per = w_mat.shape
    half = m_per // 2

    def body(x_hbm, w_hbm, out_hbm, xg_ref, xv_ref, wv_ref, wbf_ref, yv_ref,
             send_r, recv_r, send_l, recv_l, in_sems, out_sems):
        my_pos = lax.axis_index("i")
        left = (my_pos - 1) % N_DEV
        right = (my_pos + 1) % N_DEV
        opp = (my_pos + 2) % N_DEV

        cx_t = pltpu.make_async_copy(
            x_hbm.at[pl.ds(0, half)], xv_ref.at[pl.ds(0, half)],
            in_sems.at[0])
        cx_b = pltpu.make_async_copy(
            x_hbm.at[pl.ds(half, half)], xv_ref.at[pl.ds(half, half)],
            in_sems.at[1])
        cw = pltpu.make_async_copy(w_hbm, wv_ref, in_sems.at[2])
        cx_t.start()
        cx_b.start()
        cw.start()

        barrier_sem = pltpu.get_barrier_semaphore()
        for nbr in [left, right]:
            pl.semaphore_signal(
                barrier_sem, inc=1,
                device_id=(nbr,), device_id_type=pl.DeviceIdType.MESH,
            )
        pl.semaphore_wait(barrier_sem, 2)

        def copy(origin, row0, nrows, sems, slot, dst):
            return pltpu.make_async_remote_copy(
                src_ref=xg_ref.at[origin, pl.ds(row0, nrows)],
                dst_ref=xg_ref.at[origin, pl.ds(row0, nrows)],
                send_sem=sems[0].at[slot], recv_sem=sems[1].at[slot],
                device_id=(dst,), device_id_type=pl.DeviceIdType.MESH,
            )

        R = (send_r, recv_r)
        L = (send_l, recv_l)

        cx_t.wait()
        xg_ref[my_pos, pl.ds(0, half)] = \
            xv_ref[pl.ds(0, half)].astype(jnp.bfloat16)
        r0a = copy(my_pos, 0, half, R, 0, right)
        r0a.start()
        cx_b.wait()
        xg_ref[my_pos, pl.ds(half, half)] = \
            xv_ref[pl.ds(half, half)].astype(jnp.bfloat16)
        l0b = copy(my_pos, half, half, L, 0, left)
        r0b = copy(my_pos, half, half, R, 1, right)
        l0a = copy(my_pos, 0, half, L, 1, left)
        l0b.start()
        r0b.start()
        l0a.start()

        cw.wait()
        wbf_ref[:, :] = wv_ref[:, :].astype(jnp.bfloat16)

        out_copies = []

        def block_gemm(origin, row0, nrows):
            y = jnp.dot(xg_ref[origin, pl.ds(row0, nrows)], wbf_ref[:, :],
                        preferred_element_type=jnp.float32)
            rows = pl.ds(origin * m_per + row0, nrows)
            yv_ref[rows, :] = _gelu(y)
            co = pltpu.make_async_copy(
                yv_ref.at[rows], out_hbm.at[rows],
                out_sems.at[len(out_copies)])
            co.start()
            out_copies.append(co)

        def half_gemm(origin, row0):
            block_gemm(origin, row0, half)

        half_gemm(my_pos, 0)
        half_gemm(my_pos, half)

        quart = half // 2
        r0a.wait_recv()
        r1a = copy(left, 0, quart, R, 2, right)
        r1b = copy(left, quart, quart, R, 3, right)
        r1a.start()
        r1b.start()
        l0b.wait_recv()
        l1a = copy(right, half, quart, L, 2, left)
        l1b = copy(right, half + quart, quart, L, 3, left)
        l1a.start()
        l1b.start()

        half_gemm(left, 0)
        half_gemm(right, half)
        r0b.wait_recv()
        half_gemm(left, half)
        l0a.wait_recv()
        half_gemm(right, 0)

        r1a.wait_recv()
        block_gemm(opp, 0, quart)
        l1a.wait_recv()
        block_gemm(opp, half, quart)
        r1b.wait_recv()
        block_gemm(opp, quart, quart)
        l1b.wait_recv()
        block_gemm(opp, half + quart, quart)

        for c in (r0a, r0b, l0b, l0a, r1a, r1b, l1a, l1b):
            c.wait_send()
        for co in out_copies:
            co.wait()

    x = pltpu.with_memory_space_constraint(x, pltpu.MemorySpace.HBM)
    w_mat = pltpu.with_memory_space_constraint(w_mat, pltpu.MemorySpace.HBM)
    return pl.pallas_call(
        body,
        out_shape=jax.ShapeDtypeStruct((N_DEV * m_per, n_per), jnp.float32),
        in_specs=[
            pl.BlockSpec(memory_space=pltpu.MemorySpace.HBM),
            pl.BlockSpec(memory_space=pltpu.MemorySpace.HBM),
        ],
        out_specs=pl.BlockSpec(memory_space=pltpu.MemorySpace.HBM),
        scratch_shapes=[
            pltpu.VMEM((N_DEV, m_per, k), jnp.bfloat16),
            pltpu.VMEM((m_per, k), jnp.float32),
            pltpu.VMEM((k, n_per), jnp.float32),
            pltpu.VMEM((k, n_per), jnp.bfloat16),
            pltpu.VMEM((N_DEV * m_per, n_per), jnp.float32),
            pltpu.SemaphoreType.DMA((4,)),
            pltpu.SemaphoreType.DMA((4,)),
            pltpu.SemaphoreType.DMA((4,)),
            pltpu.SemaphoreType.DMA((4,)),
            pltpu.SemaphoreType.DMA((3,)),
            pltpu.SemaphoreType.DMA((10,)),
        ],
        compiler_params=pltpu.CompilerParams(collective_id=0),
    )(x, w_mat)


# device time: 42335 ns/iter; 1.0084x vs baseline; 1.0084x over previous
import jax
import jax.numpy as jnp
from jax import lax
from jax.experimental import pallas as pl
from jax.experimental.pallas import tpu as pltpu

N_DEV = 4


def _gelu(y):
    c = 0.7978845608028654
    return 0.5 * y * (1.0 + jnp.tanh(c * (y + 0.044715 * y * y * y)))


def kernel(x, w_mat):
    m_per, k = x.shape
    _, n_per = w_mat.shape
    half = m_per // 2
    quart = half // 2

    def body(x_hbm, w_hbm, out_hbm, xg_ref, xv_ref, wv_ref, wbf_ref, yv_ref,
             send_r, recv_r, send_l, recv_l, in_sems, out_sems):
        my_pos = lax.axis_index("i")
        left = (my_pos - 1) % N_DEV
        right = (my_pos + 1) % N_DEV
        opp = (my_pos + 2) % N_DEV

        def stage(row0, nrows, slot):
            c = pltpu.make_async_copy(
                x_hbm.at[pl.ds(row0, nrows)], xv_ref.at[pl.ds(row0, nrows)],
                in_sems.at[slot])
            c.start()
            return c

        cx = [stage(q * quart, quart, q) for q in range(4)]
        cw = pltpu.make_async_copy(w_hbm, wv_ref, in_sems.at[4])
        cw.start()

        barrier_sem = pltpu.get_barrier_semaphore()
        for nbr in [left, right]:
            pl.semaphore_signal(
                barrier_sem, inc=1,
                device_id=(nbr,), device_id_type=pl.DeviceIdType.MESH,
            )
        pl.semaphore_wait(barrier_sem, 2)

        def copy(origin, row0, nrows, sems, slot, dst):
            return pltpu.make_async_remote_copy(
                src_ref=xg_ref.at[origin, pl.ds(row0, nrows)],
                dst_ref=xg_ref.at[origin, pl.ds(row0, nrows)],
                send_sem=sems[0].at[slot], recv_sem=sems[1].at[slot],
                device_id=(dst,), device_id_type=pl.DeviceIdType.MESH,
            )

        R = (send_r, recv_r)
        L = (send_l, recv_l)

        def cast_quarter(q):
            cx[q].wait()
            rows = pl.ds(q * quart, quart)
            xg_ref[my_pos, rows] = xv_ref[rows].astype(jnp.bfloat16)

        cast_quarter(0)
        r0q1 = copy(my_pos, 0, quart, R, 0, right)
        r0q1.start()
        cast_quarter(2)
        l0q1 = copy(my_pos, half, quart, L, 0, left)
        l0q1.start()
        cast_quarter(1)
        r0q2 = copy(my_pos, quart, quart, R, 1, right)
        r0q2.start()
        cast_quarter(3)
        l0q2 = copy(my_pos, half + quart, quart, L, 1, left)
        l0q2.start()
        r0b = copy(my_pos, half, half, R, 2, right)
        l0a = copy(my_pos, 0, half, L, 2, left)
        r0b.start()
        l0a.start()

        cw.wait()
        wbf_ref[:, :] = wv_ref[:, :].astype(jnp.bfloat16)

        out_copies = []

        def block_gemm(origin, row0, nrows):
            y = jnp.dot(xg_ref[origin, pl.ds(row0, nrows)], wbf_ref[:, :],
                        preferred_element_type=jnp.float32)
            rows = pl.ds(origin * m_per + row0, nrows)
            yv_ref[rows, :] = _gelu(y)
            co = pltpu.make_async_copy(
                yv_ref.at[rows], out_hbm.at[rows],
                out_sems.at[len(out_copies)])
            co.start()
            out_copies.append(co)

        block_gemm(my_pos, 0, half)
        block_gemm(my_pos, half, half)

        r0q1.wait_recv()
        r0q2.wait_recv()
        r1a = copy(left, 0, quart, R, 3, right)
        r1b = copy(left, quart, quart, R, 4, right)
        r1a.start()
        r1b.start()
        l0q1.wait_recv()
        l0q2.wait_recv()
        l1a = copy(right, half, quart, L, 3, left)
        l1b = copy(right, half + quart, quart, L, 4, left)
        l1a.start()
        l1b.start()

        block_gemm(left, 0, half)
        block_gemm(right, half, half)
        r0b.wait_recv()
        block_gemm(left, half, half)
        l0a.wait_recv()
        block_gemm(right, 0, half)

        r1a.wait_recv()
        block_gemm(opp, 0, quart)
        l1a.wait_recv()
        block_gemm(opp, half, quart)
        r1b.wait_recv()
        block_gemm(opp, quart, quart)
        l1b.wait_recv()
        block_gemm(opp, half + quart, quart)

        for c in (r0q1, r0q2, r0b, l0q1, l0q2, l0a, r1a, r1b, l1a, l1b):
            c.wait_send()
        for co in out_copies:
            co.wait()

    x = pltpu.with_memory_space_constraint(x, pltpu.MemorySpace.HBM)
    w_mat = pltpu.with_memory_space_constraint(w_mat, pltpu.MemorySpace.HBM)
    return pl.pallas_call(
        body,
        out_shape=jax.ShapeDtypeStruct((N_DEV * m_per, n_per), jnp.float32),
        in_specs=[
            pl.BlockSpec(memory_space=pltpu.MemorySpace.HBM),
            pl.BlockSpec(memory_space=pltpu.MemorySpace.HBM),
        ],
        out_specs=pl.BlockSpec(memory_space=pltpu.MemorySpace.HBM),
        scratch_shapes=[
            pltpu.VMEM((N_DEV, m_per, k), jnp.bfloat16),
            pltpu.VMEM((m_per, k), jnp.float32),
            pltpu.VMEM((k, n_per), jnp.float32),
            pltpu.VMEM((k, n_per), jnp.bfloat16),
            pltpu.VMEM((N_DEV * m_per, n_per), jnp.float32),
            pltpu.SemaphoreType.DMA((5,)),
            pltpu.SemaphoreType.DMA((5,)),
            pltpu.SemaphoreType.DMA((5,)),
            pltpu.SemaphoreType.DMA((5,)),
            pltpu.SemaphoreType.DMA((5,)),
            pltpu.SemaphoreType.DMA((10,)),
        ],
        compiler_params=pltpu.CompilerParams(collective_id=0),
    )(x, w_mat)
